# baseline (speedup 1.0000x reference)
import functools

import jax
import jax.numpy as jnp
from jax import lax
from jax.experimental import pallas as pl
from jax.experimental.pallas import tpu as pltpu
from jax.experimental.pallas import tpu_sc as plsc

VOCAB_SIZE = 1000000
EMBED_DIM = 64
SEQ_LEN = 200
BATCH = 1024

NUM_WORKERS = 32
LANES = 16

CHUNK_T = 128
VOCAB_PAD = 1000064
NCHUNKS = VOCAB_PAD // CHUNK_T
CHUNKS_PER_WORKER = (NCHUNKS + NUM_WORKERS - 1) // NUM_WORKERS

BLK = 128
NBLK = BATCH // BLK
UNITS = SEQ_LEN * NBLK
UNITS_PER_WORKER = UNITS // NUM_WORKERS
GROUPS = BLK // LANES


def _repack_body(tokT_hbm, dense_hbm, c_v, d_v, sem):
    wid = lax.axis_index("s") * 2 + lax.axis_index("c")

    half_iota = lax.iota(jnp.int32, LANES) // 2
    parity64 = (lax.iota(jnp.int32, LANES) & 1) * EMBED_DIM

    def chunk(m, carry):
        k = wid + NUM_WORKERS * m

        @pl.when(k < NCHUNKS)
        def _():
            t0 = pl.multiple_of(k * CHUNK_T, CHUNK_T)
            pltpu.sync_copy(tokT_hbm.at[:, pl.ds(t0, CHUNK_T)], c_v)

            def u_group(u0, carry2):
                row_idx = u0 * (LANES // 2) + half_iota

                def a_body(a, carry3):
                    vals = c_v[a, pl.ds(u0 * LANES, LANES)]
                    col_idx = parity64 + a
                    plsc.store_scatter(d_v, [row_idx, col_idx], vals)
                    return carry3

                lax.fori_loop(0, EMBED_DIM, a_body, 0)
                return carry2

            lax.fori_loop(0, CHUNK_T // LANES, u_group, 0, unroll=True)
            d0 = pl.multiple_of(k * (CHUNK_T // 2), CHUNK_T // 2)
            pltpu.sync_copy(d_v, dense_hbm.at[pl.ds(d0, CHUNK_T // 2)])

        return carry

    lax.fori_loop(0, CHUNKS_PER_WORKER, chunk, 0)


def _gather_body(xT_hbm, tok_hbm, pos_hbm, out_hbm,
                 idx_v, rid_v, g_v, m_v, pos_v, sem):
    wid = lax.axis_index("s") * 2 + lax.axis_index("c")
    t0 = wid * UNITS_PER_WORKER

    pltpu.sync_copy(pos_hbm, pos_v)

    def run_unit(t, carry):
        s = t // NBLK
        v = t % NBLK

        pltpu.sync_copy(xT_hbm.at[s, pl.ds(v * BLK, BLK)], idx_v)

        def shift_body(i, c):
            sl = pl.ds(i * LANES, LANES)
            rid_v[sl] = lax.shift_right_logical(idx_v[sl], 1)
            return c

        lax.fori_loop(0, GROUPS, shift_body, 0, unroll=True)

        pltpu.async_copy(tok_hbm.at[rid_v], g_v, sem).wait()

        def col_group(c0, carry2):
            row_idx = lax.iota(jnp.int32, LANES) + c0 * LANES
            col_base = (idx_v[pl.ds(c0 * LANES, LANES)] & 1) * EMBED_DIM

            def a_body(a, carry3):
                pvec = plsc.load_gather(
                    pos_v, [jnp.broadcast_to(s * EMBED_DIM + a, (LANES,))])
                vals = plsc.load_gather(g_v, [row_idx, col_base + a])
                m_v[a, pl.ds(c0 * LANES, LANES)] = vals + pvec
                return carry3

            lax.fori_loop(0, EMBED_DIM, a_body, 0)
            return carry2

        lax.fori_loop(0, GROUPS, col_group, 0, unroll=True)

        pltpu.sync_copy(m_v, out_hbm.at[s, :, pl.ds(v * BLK, BLK)])
        return carry

    lax.fori_loop(t0, t0 + UNITS_PER_WORKER, run_unit, 0)


def kernel(x, token_table, pos_table):
    xT = x.T.astype(jnp.int32)
    tokT = token_table.T
    pos_flat = pos_table.reshape(SEQ_LEN * EMBED_DIM)
    mesh = plsc.VectorSubcoreMesh(core_axis_name="c", subcore_axis_name="s")

    repack = functools.partial(
        pl.kernel,
        mesh=mesh,
        out_type=jax.ShapeDtypeStruct((VOCAB_PAD // 2, 2 * EMBED_DIM),
                                      jnp.float32),
        scratch_types=[
            pltpu.VMEM((EMBED_DIM, CHUNK_T), jnp.float32),
            pltpu.VMEM((CHUNK_T // 2, 2 * EMBED_DIM), jnp.float32),
            pltpu.SemaphoreType.DMA,
        ],
        compiler_params=pltpu.CompilerParams(
            needs_layout_passes=False, disable_bounds_checks=True),
    )(_repack_body)
    dense = repack(tokT)

    gather = functools.partial(
        pl.kernel,
        mesh=mesh,
        out_type=jax.ShapeDtypeStruct((SEQ_LEN, EMBED_DIM, BATCH),
                                      jnp.float32),
        scratch_types=[
            pltpu.VMEM((BLK,), jnp.int32),
            pltpu.VMEM((BLK,), jnp.int32),
            pltpu.VMEM((BLK, 2 * EMBED_DIM), jnp.float32),
            pltpu.VMEM((EMBED_DIM, BLK), jnp.float32),
            pltpu.VMEM((SEQ_LEN * EMBED_DIM,), jnp.float32),
            pltpu.SemaphoreType.DMA,
        ],
        compiler_params=pltpu.CompilerParams(needs_layout_passes=False),
    )(_gather_body)
    out = gather(xT, dense, pos_flat)
    return out.transpose(2, 0, 1)


# speedup vs baseline: 1.2479x; 1.2479x over previous
import functools

import jax
import jax.numpy as jnp
from jax import lax
from jax.experimental import pallas as pl
from jax.experimental.pallas import tpu as pltpu
from jax.experimental.pallas import tpu_sc as plsc

VOCAB_SIZE = 1000000
EMBED_DIM = 64
SEQ_LEN = 200
BATCH = 1024

NUM_WORKERS = 32
LANES = 16

CHUNK_T = 128
VOCAB_PAD = 1000064
NCHUNKS = VOCAB_PAD // CHUNK_T
CPW = (NCHUNKS + NUM_WORKERS - 1) // NUM_WORKERS
CPW_MAIN = CPW - 1

BLK = 128
NBLK = BATCH // BLK
UNITS = SEQ_LEN * NBLK
UPW = UNITS // NUM_WORKERS
GROUPS = BLK // LANES


def _transpose_chunk(c_ref, d_ref, row_idxs, parity64):
    for u0 in range(CHUNK_T // LANES):
        for a in range(EMBED_DIM):
            vals = c_ref[a, pl.ds(u0 * LANES, LANES)]
            plsc.store_scatter(d_ref, [row_idxs[u0], parity64 + a], vals)


def _repack_body(tokT_hbm, dense_hbm, c_a, c_b, d_a, d_b,
                 gsem_a, gsem_b, ssem_a, ssem_b):
    wid = lax.axis_index("s") * 2 + lax.axis_index("c")
    c_v = (c_a, c_b)
    d_v = (d_a, d_b)
    gsem = (gsem_a, gsem_b)
    ssem = (ssem_a, ssem_b)

    half_iota = lax.iota(jnp.int32, LANES) // 2
    parity64 = (lax.iota(jnp.int32, LANES) & 1) * EMBED_DIM
    row_idxs = [half_iota + u0 * (LANES // 2)
                for u0 in range(CHUNK_T // LANES)]

    def t_of(m):
        return pl.multiple_of((wid + NUM_WORKERS * m) * CHUNK_T, CHUNK_T)

    for b in range(2):
        pltpu.async_copy(tokT_hbm.at[:, pl.ds(t_of(b), CHUNK_T)],
                         c_v[b], gsem[b])

    def body(i, carry):
        for b in range(2):
            m = 2 * i + b
            t0 = t_of(m)
            pltpu.make_async_copy(
                tokT_hbm.at[:, pl.ds(0, CHUNK_T)], c_v[b], gsem[b]).wait()

            @pl.when(i > 0)
            def _():
                pltpu.make_async_copy(
                    d_v[b], dense_hbm.at[pl.ds(0, CHUNK_T // 2)],
                    ssem[b]).wait()

            _transpose_chunk(c_v[b], d_v[b], row_idxs, parity64)

            @pl.when(m + 2 < CPW_MAIN)
            def _():
                pltpu.async_copy(
                    tokT_hbm.at[:, pl.ds(t_of(m + 2), CHUNK_T)],
                    c_v[b], gsem[b])

            d0 = pl.multiple_of(t0 // 2, CHUNK_T // 2)
            pltpu.async_copy(d_v[b], dense_hbm.at[pl.ds(d0, CHUNK_T // 2)],
                             ssem[b])
        return carry

    lax.fori_loop(0, CPW_MAIN // 2, body, 0)
    for b in range(2):
        pltpu.make_async_copy(
            d_v[b], dense_hbm.at[pl.ds(0, CHUNK_T // 2)], ssem[b]).wait()

    @pl.when(wid + NUM_WORKERS * CPW_MAIN < NCHUNKS)
    def _():
        t0 = t_of(CPW_MAIN)
        pltpu.sync_copy(tokT_hbm.at[:, pl.ds(t0, CHUNK_T)], c_a)
        _transpose_chunk(c_a, d_a, row_idxs, parity64)
        d0 = pl.multiple_of(t0 // 2, CHUNK_T // 2)
        pltpu.sync_copy(d_a, dense_hbm.at[pl.ds(d0, CHUNK_T // 2)])


def _gather_unit_compute(idx_ref, g_ref, m_ref, pos_v, s, row_idxs):
    col_bases = [(idx_ref[pl.ds(c0 * LANES, LANES)] & 1) * EMBED_DIM
                 for c0 in range(GROUPS)]
    for a in range(EMBED_DIM):
        pvec = plsc.load_gather(
            pos_v, [jnp.broadcast_to(s * EMBED_DIM + a, (LANES,))])
        for c0 in range(GROUPS):
            vals = plsc.load_gather(g_ref, [row_idxs[c0], col_bases[c0] + a])
            m_ref[a, pl.ds(c0 * LANES, LANES)] = vals + pvec


def _gather_body(xT_hbm, tok_hbm, pos_hbm, out_hbm,
                 idx_a, idx_b, rid_a, rid_b, g_a, g_b, m_a, m_b, pos_v,
                 gsem_a, gsem_b, ssem_a, ssem_b):
    wid = lax.axis_index("s") * 2 + lax.axis_index("c")
    t0 = wid * UPW
    idx_v = (idx_a, idx_b)
    rid_v = (rid_a, rid_b)
    g_v = (g_a, g_b)
    m_v = (m_a, m_b)
    gsem = (gsem_a, gsem_b)
    ssem = (ssem_a, ssem_b)

    pltpu.sync_copy(pos_hbm, pos_v)

    row_idxs = [lax.iota(jnp.int32, LANES) + c0 * LANES
                for c0 in range(GROUPS)]

    def launch(t, b):
        s = t // NBLK
        v = t % NBLK
        pltpu.sync_copy(xT_hbm.at[s, pl.ds(v * BLK, BLK)], idx_v[b])
        for i in range(GROUPS):
            sl = pl.ds(i * LANES, LANES)
            rid_v[b][sl] = lax.shift_right_logical(idx_v[b][sl], 1)
        pltpu.async_copy(tok_hbm.at[rid_v[b]], g_v[b], gsem[b])

    for b in range(2):
        launch(t0 + b, b)

    def body(j, carry):
        for b in range(2):
            t = t0 + 2 * j + b
            s = t // NBLK
            v = t % NBLK
            pltpu.make_async_copy(
                tok_hbm.at[rid_v[b]], g_v[b], gsem[b]).wait()

            @pl.when(j > 0)
            def _():
                pltpu.make_async_copy(
                    m_v[b], out_hbm.at[0, :, pl.ds(0, BLK)], ssem[b]).wait()

            _gather_unit_compute(idx_v[b], g_v[b], m_v[b], pos_v, s,
                                 row_idxs)

            @pl.when(2 * j + b + 2 < UPW)
            def _():
                launch(t + 2, b)

            pltpu.async_copy(m_v[b], out_hbm.at[s, :, pl.ds(v * BLK, BLK)],
                             ssem[b])
        return carry

    lax.fori_loop(0, UPW // 2, body, 0)
    for b in range(2):
        pltpu.make_async_copy(
            m_v[b], out_hbm.at[0, :, pl.ds(0, BLK)], ssem[b]).wait()


def kernel(x, token_table, pos_table):
    xT = x.T.astype(jnp.int32)
    tokT = token_table.T
    pos_flat = pos_table.reshape(SEQ_LEN * EMBED_DIM)
    mesh = plsc.VectorSubcoreMesh(core_axis_name="c", subcore_axis_name="s")

    repack = functools.partial(
        pl.kernel,
        mesh=mesh,
        out_type=jax.ShapeDtypeStruct((VOCAB_PAD // 2, 2 * EMBED_DIM),
                                      jnp.float32),
        scratch_types=[
            pltpu.VMEM((EMBED_DIM, CHUNK_T), jnp.float32),
            pltpu.VMEM((EMBED_DIM, CHUNK_T), jnp.float32),
            pltpu.VMEM((CHUNK_T // 2, 2 * EMBED_DIM), jnp.float32),
            pltpu.VMEM((CHUNK_T // 2, 2 * EMBED_DIM), jnp.float32),
            pltpu.SemaphoreType.DMA,
            pltpu.SemaphoreType.DMA,
            pltpu.SemaphoreType.DMA,
            pltpu.SemaphoreType.DMA,
        ],
        compiler_params=pltpu.CompilerParams(
            needs_layout_passes=False, disable_bounds_checks=True),
    )(_repack_body)
    dense = repack(tokT)

    gather = functools.partial(
        pl.kernel,
        mesh=mesh,
        out_type=jax.ShapeDtypeStruct((SEQ_LEN, EMBED_DIM, BATCH),
                                      jnp.float32),
        scratch_types=[
            pltpu.VMEM((BLK,), jnp.int32),
            pltpu.VMEM((BLK,), jnp.int32),
            pltpu.VMEM((BLK,), jnp.int32),
            pltpu.VMEM((BLK,), jnp.int32),
            pltpu.VMEM((BLK, 2 * EMBED_DIM), jnp.float32),
            pltpu.VMEM((BLK, 2 * EMBED_DIM), jnp.float32),
            pltpu.VMEM((EMBED_DIM, BLK), jnp.float32),
            pltpu.VMEM((EMBED_DIM, BLK), jnp.float32),
            pltpu.VMEM((SEQ_LEN * EMBED_DIM,), jnp.float32),
            pltpu.SemaphoreType.DMA,
            pltpu.SemaphoreType.DMA,
            pltpu.SemaphoreType.DMA,
            pltpu.SemaphoreType.DMA,
        ],
        compiler_params=pltpu.CompilerParams(needs_layout_passes=False),
    )(_gather_body)
    out = gather(xT, dense, pos_flat)
    return out.transpose(2, 0, 1)
